# baseline (speedup 1.0000x reference)
import functools

import jax
import jax.numpy as jnp
from jax import lax
from jax.experimental import pallas as pl
from jax.experimental.pallas import tpu as pltpu
from jax.experimental.pallas import tpu_sc as plsc

_DIM = 16
_B = 16384
_H = 50
_MEAN = 3.0
_STD = 1.25

_NC = 2
_NS = 16
_NW = _NC * _NS
_BPW = _B // _NW
_CB = 64
_NCH = _BPW // _CB
_IG = 128
_GPC = _CB * _H // _IG
_UG = _BPW // _IG


def _sc_body(rating_hbm, uidx_hbm, hidx_hbm, hrow_hbm, utab_hbm, htab_hbm,
             r_out, u_out, h_out,
             hidx0, hidx1, hcnt0, hcnt1, hrows0, hrows1,
             urows_v, rating_v, pooled_v, t0_v, sem0, sem1):
  wid = lax.axis_index("s") * _NC + lax.axis_index("c")
  base = wid * _BPW
  hidx = (hidx0, hidx1)
  hcnt = (hcnt0, hcnt1)
  hrows = (hrows0, hrows1)
  sems = (sem0, sem1)
  iot = lax.broadcasted_iota(jnp.int32, (16,), 0)

  pltpu.sync_copy(htab_hbm.at[pl.ds(0, 1)], t0_v.at[pl.ds(0, 1)])
  t0 = t0_v[...].astype(jnp.float32)[0, :]

  uidx_view = hidx1.at[pl.ds(0, _BPW)]
  pltpu.sync_copy(uidx_hbm.at[pl.ds(base, _BPW)], uidx_view)
  ucopies = [
      pltpu.make_async_copy(utab_hbm.at[uidx_view.at[pl.ds(k * _IG, _IG)]],
                            urows_v.at[pl.ds(k * _IG, _IG)], sem1)
      for k in range(_UG)
  ]
  for cp in ucopies:
    cp.start()

  def stage_and_fire(c, par):
    rb = base + c * _CB
    pltpu.sync_copy(hidx_hbm.at[pl.ds(rb * _H, _CB * _H)], hidx[par])
    cnt_cp = pltpu.make_async_copy(hrow_hbm.at[pl.ds(rb, _CB)], hcnt[par],
                                   sems[par])
    cnt_cp.start()

    def fire(g, carry):
      pltpu.make_async_copy(
          htab_hbm.at[hidx[par].at[pl.ds(g * _IG, _IG)]],
          hrows[par].at[pl.ds(g * _IG, _IG)], sems[par]).start()
      return carry

    lax.fori_loop(0, _GPC, fire, 0)
    return cnt_cp

  cnt_cp = stage_and_fire(0, 0)

  pltpu.sync_copy(rating_hbm.at[pl.ds(base, _BPW)], rating_v)
  for k in range(_BPW // 16):
    rating_v[pl.ds(k * 16, 16)] = (
        rating_v[pl.ds(k * 16, 16)] - _MEAN) / _STD
  pltpu.sync_copy(rating_v, r_out.at[pl.ds(base, _BPW)])

  for cp in ucopies:
    cp.wait()
  pltpu.sync_copy(urows_v, u_out.at[pl.ds(base, _BPW)])

  for c in range(_NCH):
    par = c % 2
    if c + 1 < _NCH:
      next_cnt_cp = stage_and_fire(c + 1, (c + 1) % 2)
    cnt_cp.wait()

    def drain(g, carry):
      pltpu.make_async_copy(
          htab_hbm.at[hidx[par].at[pl.ds(0, _IG)]],
          hrows[par].at[pl.ds(0, _IG)], sems[par]).wait()
      return carry

    lax.fori_loop(0, _GPC, drain, 0)

    def comp(b, carry):
      fb = b * _H
      a0 = hrows[par][pl.ds(fb, 2), :].astype(jnp.float32)
      a1 = hrows[par][pl.ds(fb + 2, 2), :].astype(jnp.float32)
      for j in range(2, _H // 2):
        pair = hrows[par][pl.ds(fb + 2 * j, 2), :].astype(jnp.float32)
        if j % 2 == 0:
          a0 = a0 + pair
        else:
          a1 = a1 + pair
      a = a0 + a1
      acc = a[0, :] + a[1, :]
      r0 = hcnt[par][b, pl.ds(0, 16)]
      r1 = hcnt[par][b, pl.ds(16, 16)]
      r2 = hcnt[par][b, pl.ds(32, 16)]
      r3 = hcnt[par][b, pl.ds(34, 16)]
      z = (jnp.where(r0 == 0, 1.0, 0.0) + jnp.where(r1 == 0, 1.0, 0.0) +
           jnp.where(r2 == 0, 1.0, 0.0) +
           jnp.where((r3 == 0) & (iot >= 14), 1.0, 0.0))
      for k in (8, 4, 2, 1):
        z = z + z.at[(iot + k) & 15].get(mode="promise_in_bounds")
      n0 = z
      denom = jnp.maximum(_H - n0, 1e-9)
      pooled_v[c * _CB + b, :] = (acc - n0 * t0) / denom
      return carry

    lax.fori_loop(0, _CB, comp, 0)
    if c + 1 < _NCH:
      cnt_cp = next_cnt_cp

  pltpu.sync_copy(pooled_v, h_out.at[pl.ds(base, _BPW)])


def kernel(user_rating, user_id_idx, history_idx, user_id_table, history_table):
  b = user_rating.shape[0]
  f32 = jnp.float32
  mesh = plsc.VectorSubcoreMesh(core_axis_name="c", subcore_axis_name="s")
  run = functools.partial(
      pl.kernel,
      out_type=[
          jax.ShapeDtypeStruct((b,), f32),
          jax.ShapeDtypeStruct((b, _DIM), f32),
          jax.ShapeDtypeStruct((b, _DIM), f32),
      ],
      mesh=mesh,
      compiler_params=pltpu.CompilerParams(use_tc_tiling_on_sc=False),
      scratch_types=[
          pltpu.VMEM((_CB * _H,), jnp.int32),
          pltpu.VMEM((_CB * _H,), jnp.int32),
          pltpu.VMEM((_CB, _H), jnp.int32),
          pltpu.VMEM((_CB, _H), jnp.int32),
          pltpu.VMEM((_CB * _H, _DIM), jnp.bfloat16),
          pltpu.VMEM((_CB * _H, _DIM), jnp.bfloat16),
          pltpu.VMEM((_BPW, _DIM), f32),
          pltpu.VMEM((_BPW,), f32),
          pltpu.VMEM((_BPW, _DIM), f32),
          pltpu.VMEM((2, _DIM), jnp.bfloat16),
          pltpu.SemaphoreType.DMA,
          pltpu.SemaphoreType.DMA,
      ],
  )(_sc_body)
  hflat = history_idx.reshape(b * _H)
  r, u, h = run(user_rating, user_id_idx, hflat, history_idx,
                user_id_table, history_table.astype(jnp.bfloat16))
  return jnp.concatenate([r[:, None], u, h], axis=-1)


# speedup vs baseline: 1.1088x; 1.1088x over previous
import functools

import jax
import jax.numpy as jnp
from jax import lax
from jax.experimental import pallas as pl
from jax.experimental.pallas import tpu as pltpu
from jax.experimental.pallas import tpu_sc as plsc

_DIM = 16
_B = 16384
_H = 50
_MEAN = 3.0
_STD = 1.25

_NC = 2
_NS = 16
_NW = _NC * _NS
_BPW = _B // _NW
_CB = 64
_NCH = _BPW // _CB
_IG = 128
_GPC = _CB * _H // _IG
_UG = _BPW // _IG


def _sc_body(rating_hbm, uidx_hbm, hidx_hbm, hrow_hbm, utab_hbm, htab_hbm,
             r_out, u_out, h_out,
             hidx0, hidx1, hcnt0, hcnt1, hrows0, hrows1,
             rating_v, pooled_v, t0_v, sem0, sem1):
  wid = lax.axis_index("s") * _NC + lax.axis_index("c")
  base = wid * _BPW
  hidx = (hidx0, hidx1)
  hcnt = (hcnt0, hcnt1)
  hrows = (hrows0, hrows1)
  sems = (sem0, sem1)

  pltpu.sync_copy(htab_hbm.at[0], t0_v)
  t0 = t0_v[...]

  uidx_view = hidx1.at[pl.ds(0, _BPW)]
  pltpu.sync_copy(uidx_hbm.at[pl.ds(base, _BPW)], uidx_view)
  ucopies = [
      pltpu.make_async_copy(utab_hbm.at[uidx_view.at[pl.ds(k * _IG, _IG)]],
                            hrows1.at[pl.ds(k * _IG, _IG)], sem1)
      for k in range(_UG)
  ]
  for cp in ucopies:
    cp.start()

  def stage_and_fire(c, par):
    rb = base + c * _CB
    pltpu.sync_copy(hidx_hbm.at[pl.ds(rb * _H, _CB * _H)], hidx[par])
    cnt_cp = pltpu.make_async_copy(hrow_hbm.at[pl.ds(rb, _CB)], hcnt[par],
                                   sems[par])
    cnt_cp.start()

    pltpu.make_async_copy(htab_hbm.at[hidx[par]], hrows[par],
                          sems[par]).start()
    return cnt_cp

  cnt_cp = stage_and_fire(0, 0)

  pltpu.sync_copy(rating_hbm.at[pl.ds(base, _BPW)], rating_v)
  for k in range(_BPW // 16):
    rating_v[pl.ds(k * 16, 16)] = (
        rating_v[pl.ds(k * 16, 16)] - _MEAN) / _STD
  pltpu.sync_copy(rating_v, r_out.at[pl.ds(base, _BPW)])

  for cp in ucopies:
    cp.wait()
  pltpu.sync_copy(hrows1.at[pl.ds(0, _BPW)], u_out.at[pl.ds(base, _BPW)])

  iot = lax.broadcasted_iota(jnp.int32, (16,), 0)

  for c in range(_NCH):
    par = c % 2
    if c + 1 < _NCH:
      next_cnt_cp = stage_and_fire(c + 1, (c + 1) % 2)
    cnt_cp.wait()

    pltpu.make_async_copy(htab_hbm.at[hidx[par]], hrows[par],
                          sems[par]).wait()

    def comp(b, carry):
      fb = b * _H
      accs = [hrows[par][fb + h, :] for h in range(4)]
      for h in range(4, _H):
        accs[h % 4] = accs[h % 4] + hrows[par][fb + h, :]
      acc = (accs[0] + accs[1]) + (accs[2] + accs[3])
      r0 = hcnt[par][b, pl.ds(0, 16)]
      r1 = hcnt[par][b, pl.ds(16, 16)]
      r2 = hcnt[par][b, pl.ds(32, 16)]
      r3 = hcnt[par][b, pl.ds(34, 16)]
      z = (jnp.where(r0 == 0, 1.0, 0.0) + jnp.where(r1 == 0, 1.0, 0.0) +
           jnp.where(r2 == 0, 1.0, 0.0) +
           jnp.where((r3 == 0) & (iot >= 14), 1.0, 0.0))
      for k in (8, 4, 2, 1):
        z = z + z.at[(iot + k) & 15].get(mode="promise_in_bounds")
      n0 = z
      denom = jnp.maximum(_H - n0, 1e-9)
      pooled_v[c * _CB + b, :] = (acc - n0 * t0) / denom
      return carry

    lax.fori_loop(0, _CB, comp, 0)
    if c + 1 < _NCH:
      cnt_cp = next_cnt_cp

  pltpu.sync_copy(pooled_v, h_out.at[pl.ds(base, _BPW)])


def kernel(user_rating, user_id_idx, history_idx, user_id_table, history_table):
  b = user_rating.shape[0]
  f32 = jnp.float32
  mesh = plsc.VectorSubcoreMesh(core_axis_name="c", subcore_axis_name="s")
  run = functools.partial(
      pl.kernel,
      out_type=[
          jax.ShapeDtypeStruct((b,), f32),
          jax.ShapeDtypeStruct((b, _DIM), f32),
          jax.ShapeDtypeStruct((b, _DIM), f32),
      ],
      mesh=mesh,
      compiler_params=pltpu.CompilerParams(use_tc_tiling_on_sc=False),
      scratch_types=[
          pltpu.VMEM((_CB * _H,), jnp.int32),
          pltpu.VMEM((_CB * _H,), jnp.int32),
          pltpu.VMEM((_CB, _H), jnp.int32),
          pltpu.VMEM((_CB, _H), jnp.int32),
          pltpu.VMEM((_CB * _H, _DIM), f32),
          pltpu.VMEM((_CB * _H, _DIM), f32),
          pltpu.VMEM((_BPW,), f32),
          pltpu.VMEM((_BPW, _DIM), f32),
          pltpu.VMEM((_DIM,), f32),
          pltpu.SemaphoreType.DMA,
          pltpu.SemaphoreType.DMA,
      ],
  )(_sc_body)
  hflat = history_idx.reshape(b * _H)
  r, u, h = run(user_rating, user_id_idx, hflat, history_idx,
                user_id_table, history_table)
  return jnp.concatenate([r[:, None], u, h], axis=-1)


# speedup vs baseline: 1.1091x; 1.0003x over previous
import functools

import jax
import jax.numpy as jnp
from jax import lax
from jax.experimental import pallas as pl
from jax.experimental.pallas import tpu as pltpu
from jax.experimental.pallas import tpu_sc as plsc

_DIM = 16
_B = 16384
_H = 50
_MEAN = 3.0
_STD = 1.25

_NC = 2
_NS = 16
_NW = _NC * _NS
_BPW = _B // _NW
_CB = 64
_NCH = _BPW // _CB
_IG = 128
_UG = _BPW // _IG


def _sc_body(rating_hbm, uidx_hbm, hidx_hbm, hrow_hbm, utab_hbm, htab_hbm,
             r_out, u_out, h_out,
             hidx0, hidx1, hcnt0, hcnt1, hrows0, hrows1,
             rating_v, pooled_v, t0_v, sem0, sem1):
  wid = lax.axis_index("s") * _NC + lax.axis_index("c")
  base = wid * _BPW
  hidx = (hidx0, hidx1)
  hcnt = (hcnt0, hcnt1)
  hrows = (hrows0, hrows1)
  sems = (sem0, sem1)

  pltpu.sync_copy(htab_hbm.at[0], t0_v)
  t0 = t0_v[...]

  uidx_view = hidx1.at[pl.ds(0, _BPW)]
  pltpu.sync_copy(uidx_hbm.at[pl.ds(base, _BPW)], uidx_view)
  ucopies = [
      pltpu.make_async_copy(utab_hbm.at[uidx_view.at[pl.ds(k * _IG, _IG)]],
                            hrows1.at[pl.ds(k * _IG, _IG)], sem1)
      for k in range(_UG)
  ]
  for cp in ucopies:
    cp.start()

  def stage_and_fire(c, par):
    rb = base + c * _CB
    pltpu.sync_copy(hidx_hbm.at[pl.ds(rb * _H, _CB * _H)], hidx[par])
    cnt_cp = pltpu.make_async_copy(hrow_hbm.at[pl.ds(rb, _CB)], hcnt[par],
                                   sems[par])
    cnt_cp.start()

    pltpu.make_async_copy(htab_hbm.at[hidx[par]], hrows[par],
                          sems[par]).start()
    return cnt_cp

  cnt_cp = stage_and_fire(0, 0)

  pltpu.sync_copy(rating_hbm.at[pl.ds(base, _BPW)], rating_v)
  for k in range(_BPW // 16):
    rating_v[pl.ds(k * 16, 16)] = (
        rating_v[pl.ds(k * 16, 16)] - _MEAN) / _STD
  pltpu.sync_copy(rating_v, r_out.at[pl.ds(base, _BPW)])

  for cp in ucopies:
    cp.wait()
  pltpu.sync_copy(hrows1.at[pl.ds(0, _BPW)], u_out.at[pl.ds(base, _BPW)])

  iot = lax.broadcasted_iota(jnp.int32, (16,), 0)

  for c in range(_NCH):
    par = c % 2
    if c + 1 < _NCH:
      next_cnt_cp = stage_and_fire(c + 1, (c + 1) % 2)
    cnt_cp.wait()

    pltpu.make_async_copy(htab_hbm.at[hidx[par]], hrows[par],
                          sems[par]).wait()

    def comp(b, carry):
      fb = b * _H
      accs = [hrows[par][fb + h, :] for h in range(4)]
      for h in range(4, _H):
        accs[h % 4] = accs[h % 4] + hrows[par][fb + h, :]
      acc = (accs[0] + accs[1]) + (accs[2] + accs[3])
      r0 = hcnt[par][b, pl.ds(0, 16)]
      r1 = hcnt[par][b, pl.ds(16, 16)]
      r2 = hcnt[par][b, pl.ds(32, 16)]
      r3 = hcnt[par][b, pl.ds(34, 16)]
      z = (jnp.where(r0 == 0, 1.0, 0.0) + jnp.where(r1 == 0, 1.0, 0.0) +
           jnp.where(r2 == 0, 1.0, 0.0) +
           jnp.where((r3 == 0) & (iot >= 14), 1.0, 0.0))
      for k in (8, 4, 2, 1):
        z = z + z.at[(iot + k) & 15].get(mode="promise_in_bounds")
      n0 = z
      denom = jnp.maximum(_H - n0, 1e-9)
      pooled_v[c * _CB + b, :] = (acc - n0 * t0) / denom
      return carry

    lax.fori_loop(0, _CB, comp, 0)
    if c + 1 < _NCH:
      cnt_cp = next_cnt_cp

  pltpu.sync_copy(pooled_v, h_out.at[pl.ds(base, _BPW)])


def kernel(user_rating, user_id_idx, history_idx, user_id_table, history_table):
  b = user_rating.shape[0]
  f32 = jnp.float32
  mesh = plsc.VectorSubcoreMesh(core_axis_name="c", subcore_axis_name="s")
  run = functools.partial(
      pl.kernel,
      out_type=[
          jax.ShapeDtypeStruct((b,), f32),
          jax.ShapeDtypeStruct((b, _DIM), f32),
          jax.ShapeDtypeStruct((b, _DIM), f32),
      ],
      mesh=mesh,
      compiler_params=pltpu.CompilerParams(use_tc_tiling_on_sc=False),
      scratch_types=[
          pltpu.VMEM((_CB * _H,), jnp.int32),
          pltpu.VMEM((_CB * _H,), jnp.int32),
          pltpu.VMEM((_CB, _H), jnp.int32),
          pltpu.VMEM((_CB, _H), jnp.int32),
          pltpu.VMEM((_CB * _H, _DIM), f32),
          pltpu.VMEM((_CB * _H, _DIM), f32),
          pltpu.VMEM((_BPW,), f32),
          pltpu.VMEM((_BPW, _DIM), f32),
          pltpu.VMEM((_DIM,), f32),
          pltpu.SemaphoreType.DMA,
          pltpu.SemaphoreType.DMA,
      ],
  )(_sc_body)
  hflat = history_idx.reshape(b * _H)
  r, u, h = run(user_rating, user_id_idx, hflat, history_idx,
                user_id_table, history_table)
  return jnp.concatenate([r[:, None], u, h], axis=-1)
